# Initial kernel scaffold; baseline (speedup 1.0000x reference)
import jax
import jax.numpy as jnp
from jax.experimental import pallas as pl


def kernel(input, weights):
    raise NotImplementedError("write your pallas kernel here")


# speedup vs baseline: 1.7393x; 1.7393x over previous
import functools

import jax
import jax.numpy as jnp
from jax import lax
from jax.experimental import pallas as pl
from jax.experimental.pallas import tpu as pltpu
from jax.experimental.pallas import tpu_sc as plsc

_PAD = 1
_B = 4
_S = 8192
_D = 1024
_TOT = _B * _S

_NC = 2
_NS = 16
_NW = _NC * _NS
_PER_W = _TOT // _NW
_CH = 32
_NCHUNK = _PER_W // _CH
_NBUF = 2

_LANES = 128
_SUB = _S // _LANES
_R = _B * _SUB


def _pos_body(x_ref, out_ref):
    mf = (x_ref[...] != _PAD).astype(jnp.float32)
    r128 = lax.broadcasted_iota(jnp.int32, (_LANES, _LANES), 0)
    c128 = lax.broadcasted_iota(jnp.int32, (_LANES, _LANES), 1)
    upper = (r128 <= c128).astype(jnp.float32)
    within = jnp.dot(mf, upper, preferred_element_type=jnp.float32)
    rs = jnp.broadcast_to(within[:, _LANES - 1:_LANES], (_R, _LANES))
    rr = lax.broadcasted_iota(jnp.int32, (_R, _R), 0)
    cc = lax.broadcasted_iota(jnp.int32, (_R, _R), 1)
    lower = ((cc < rr) & (rr // _SUB == cc // _SUB)).astype(jnp.float32)
    off = jnp.dot(lower, rs, preferred_element_type=jnp.float32)
    pos = (within + off) * mf + float(_PAD)
    out_ref[...] = pos.astype(jnp.int32)


def _positions(x):
    x2 = x.reshape(_R, _LANES)
    pos = pl.pallas_call(
        _pos_body,
        out_shape=jax.ShapeDtypeStruct((_R, _LANES), jnp.int32),
    )(x2)
    return pos.reshape(_NW, _NCHUNK, _CH)


def _sc_gather_body(table_hbm, idx_hbm, out_hbm, idx_v, bufs, gsem):
    wid = lax.axis_index("s") * _NC + lax.axis_index("c")
    base = wid * _PER_W
    pltpu.sync_copy(idx_hbm.at[wid], idx_v)

    def g_desc(ci, b):
        return pltpu.make_async_copy(
            table_hbm.at[idx_v.at[ci]], bufs.at[b], gsem.at[b])

    for b in range(_NBUF):
        g_desc(b, b).start()

    def round_body(g, carry):
        for b in range(_NBUF):
            ci = g * _NBUF + b
            g_desc(ci, b).wait()
            pltpu.sync_copy(bufs.at[b], out_hbm.at[pl.ds(base + ci * _CH, _CH)])
            g_desc(ci + _NBUF, b).start()
        return carry

    rounds = (_NCHUNK - _NBUF) // _NBUF
    lax.fori_loop(0, rounds, round_body, 0)

    for b in range(_NBUF):
        ci = _NCHUNK - _NBUF + b
        g_desc(ci, b).wait()
        pltpu.sync_copy(bufs.at[b], out_hbm.at[pl.ds(base + ci * _CH, _CH)])


@functools.cache
def _sc_gather():
    return pl.kernel(
        _sc_gather_body,
        out_type=jax.ShapeDtypeStruct((_TOT, _D), jnp.float32),
        mesh=plsc.VectorSubcoreMesh(
            core_axis_name="c", subcore_axis_name="s",
            num_cores=_NC, num_subcores=_NS),
        scratch_types=[
            pltpu.VMEM((_NCHUNK, _CH), jnp.int32),
            pltpu.VMEM((_NBUF, _CH, _D), jnp.float32),
            pltpu.SemaphoreType.DMA((_NBUF,)),
        ],
    )


def kernel(input, weights):
    pos = _positions(input.astype(jnp.int32))
    flat = _sc_gather()(weights, pos)
    return flat.reshape(_B, _S, _D)
